# Initial kernel scaffold; baseline (speedup 1.0000x reference)
import jax
import jax.numpy as jnp
from jax.experimental import pallas as pl


def kernel(x, features, W, b):
    raise NotImplementedError("write your pallas kernel here")


# speedup vs baseline: 7.5237x; 7.5237x over previous
import functools

import jax
import jax.numpy as jnp
from jax import lax
from jax.experimental import pallas as pl
from jax.experimental.pallas import tpu as pltpu
from jax.experimental.pallas import tpu_sc as plsc

B, N, D = 16, 1024, 64
K = 8
D_MODEL = 512
D2 = 2 * D

_ROWS = B * N * K
_FF_TILE = 512


def _prep_body(x_ref, f_ref, xn_ref, gidx_ref):
    b = pl.program_id(0)
    xm = x_ref[0]
    fr = f_ref[0]
    mask = fr > 0.1
    x_crd = jnp.where(mask, 0.0, xm)
    x_ftr = jnp.where(mask, xm, 0.0)
    xc = jnp.concatenate([x_crd, x_ftr], axis=1)
    mean = jnp.mean(xc, axis=0, keepdims=True)
    cent = xc - mean
    var = jnp.sum(cent * cent, axis=0, keepdims=True) / (N - 1)
    std = jnp.sqrt(var)
    xn_ref[0] = jnp.clip(cent / (std + 1e-5), -10.0, 10.0)

    sq = jnp.sum(x_crd * x_crd, axis=1, keepdims=True)
    xb = x_crd.astype(jnp.bfloat16)
    dot = lax.dot_general(xb, xb, (((1,), (1,)), ((), ())),
                          preferred_element_type=jnp.float32)
    ones = jnp.ones((1, D), jnp.float32)
    xsq = x_crd * x_crd
    sq_row = lax.dot_general(ones, xsq, (((1,), (1,)), ((), ())),
                             preferred_element_type=jnp.float32,
                             precision=lax.Precision.HIGHEST)
    d2 = sq + sq_row - 2.0 * dot
    dist = jnp.sqrt(jnp.maximum(d2, 1e-12))

    col = lax.broadcasted_iota(jnp.int32, (N, N), 1)
    work = dist
    idxs = []
    for _ in range(K):
        m = jnp.min(work, axis=1, keepdims=True)
        cand = jnp.where(work == m, col, N)
        ik = jnp.min(cand, axis=1, keepdims=True)
        idxs.append(ik)
        work = jnp.where(col == ik, jnp.inf, work)
    gidx_ref[0] = jnp.concatenate(idxs, axis=1) + b * N


def _prep_call(x, features):
    f3 = features.reshape(B, 1, D)
    return pl.pallas_call(
        _prep_body,
        grid=(B,),
        in_specs=[
            pl.BlockSpec((1, N, D), lambda b: (b, 0, 0)),
            pl.BlockSpec((1, 1, D), lambda b: (b, 0, 0)),
        ],
        out_specs=[
            pl.BlockSpec((1, N, D2), lambda b: (b, 0, 0)),
            pl.BlockSpec((1, N, K), lambda b: (b, 0, 0)),
        ],
        out_shape=[
            jax.ShapeDtypeStruct((B, N, D2), jnp.float32),
            jax.ShapeDtypeStruct((B, N, K), jnp.int32),
        ],
    )(x, f3)


_CHUNK = 128


def _gather_call(table, gidx_flat):
    info = plsc.get_sparse_core_info()
    num_cores = info.num_cores
    nw = num_cores * info.num_subcores
    rpw = _ROWS // nw
    nchunk = rpw // _CHUNK
    mesh = plsc.VectorSubcoreMesh(core_axis_name="c", subcore_axis_name="s")

    @functools.partial(
        pl.kernel,
        mesh=mesh,
        out_type=jax.ShapeDtypeStruct((_ROWS, D2), jnp.float32),
        scratch_types=[
            pltpu.VMEM((_CHUNK,), jnp.int32),
            pltpu.VMEM((_CHUNK, D2), jnp.float32),
            pltpu.SemaphoreType.DMA,
        ],
    )
    def gather_kernel(table_hbm, idx_hbm, out_hbm, idx_v, rows_v, sem):
        wid = lax.axis_index("s") * num_cores + lax.axis_index("c")
        base = wid * rpw

        def body(c, carry):
            off = base + c * _CHUNK
            pltpu.sync_copy(idx_hbm.at[pl.ds(off, _CHUNK)], idx_v)
            pltpu.async_copy(table_hbm.at[idx_v], rows_v, sem).wait()
            pltpu.sync_copy(rows_v, out_hbm.at[pl.ds(off, _CHUNK)])
            return carry

        lax.fori_loop(0, nchunk, body, 0)

    return gather_kernel(table, gidx_flat)


def _ff_body(g_ref, xn_ref, w_ref, b_ref, o_ref):
    wm = w_ref[...]
    ws = wm[:, 0:D2]
    for k in range(1, K):
        ws = ws + wm[:, k * D2:(k + 1) * D2]
    h = lax.dot_general(g_ref[...], wm, (((1,), (1,)), ((), ())),
                        preferred_element_type=jnp.float32,
                        precision=lax.Precision.HIGHEST)
    h = h - lax.dot_general(xn_ref[...], ws, (((1,), (1,)), ((), ())),
                            preferred_element_type=jnp.float32,
                            precision=lax.Precision.HIGHEST)
    h = h + b_ref[...]
    a = h[:, :D_MODEL]
    g = h[:, D_MODEL:]
    o_ref[...] = a * jax.nn.sigmoid(g)


def _ff_call(g2, xnf, W, b):
    brow = b.reshape(1, 2 * D_MODEL)
    n_tiles = (B * N) // _FF_TILE
    return pl.pallas_call(
        _ff_body,
        grid=(n_tiles,),
        in_specs=[
            pl.BlockSpec((_FF_TILE, K * D2), lambda i: (i, 0)),
            pl.BlockSpec((_FF_TILE, D2), lambda i: (i, 0)),
            pl.BlockSpec((2 * D_MODEL, K * D2), lambda i: (0, 0)),
            pl.BlockSpec((1, 2 * D_MODEL), lambda i: (0, 0)),
        ],
        out_specs=pl.BlockSpec((_FF_TILE, D_MODEL), lambda i: (i, 0)),
        out_shape=jax.ShapeDtypeStruct((B * N, D_MODEL), jnp.float32),
    )(g2, xnf, W, brow)


def kernel(x, features, W, b):
    xn, gidx = _prep_call(x, features)
    table = xn.reshape(B * N, D2)
    gidx_flat = gidx.reshape(_ROWS)
    g = _gather_call(table, gidx_flat)
    g2 = g.reshape(B * N, K * D2)
    out = _ff_call(g2, table, W, b)
    return out.reshape(B, N, D_MODEL)


# speedup vs baseline: 11.1575x; 1.4830x over previous
import functools

import jax
import jax.numpy as jnp
from jax import lax
from jax.experimental import pallas as pl
from jax.experimental.pallas import tpu as pltpu
from jax.experimental.pallas import tpu_sc as plsc

B, N, D = 16, 1024, 64
K = 8
D_MODEL = 512
D2 = 2 * D

_ROWS = B * N * K
_FF_TILE = 512


def _prep_body(x_ref, f_ref, xn_ref, gidx_ref):
    b = pl.program_id(0)
    xm = x_ref[0]
    fr = f_ref[0]
    mask = fr > 0.1
    x_crd = jnp.where(mask, 0.0, xm)
    x_ftr = jnp.where(mask, xm, 0.0)
    xc = jnp.concatenate([x_crd, x_ftr], axis=1)
    mean = jnp.mean(xc, axis=0, keepdims=True)
    cent = xc - mean
    var = jnp.sum(cent * cent, axis=0, keepdims=True) / (N - 1)
    std = jnp.sqrt(var)
    xn_ref[0] = jnp.clip(cent / (std + 1e-5), -10.0, 10.0)

    sq = jnp.sum(x_crd * x_crd, axis=1, keepdims=True)
    xb = x_crd.astype(jnp.bfloat16)
    dot = lax.dot_general(xb, xb, (((1,), (1,)), ((), ())),
                          preferred_element_type=jnp.float32)
    ones = jnp.ones((1, D), jnp.float32)
    xsq = x_crd * x_crd
    sq_row = lax.dot_general(ones, xsq, (((1,), (1,)), ((), ())),
                             preferred_element_type=jnp.float32,
                             precision=lax.Precision.HIGHEST)
    d2 = sq + sq_row - 2.0 * dot
    dist = jnp.sqrt(jnp.maximum(d2, 1e-12))

    col = lax.broadcasted_iota(jnp.int32, (N, N), 1)
    work = dist
    idxs = []
    for _ in range(K):
        m = jnp.min(work, axis=1, keepdims=True)
        cand = jnp.where(work == m, col, N)
        ik = jnp.min(cand, axis=1, keepdims=True)
        idxs.append(ik)
        work = jnp.where(col == ik, jnp.inf, work)
    gidx_ref[0] = jnp.concatenate(idxs, axis=1) + b * N


def _prep_call(x, features):
    f3 = features.reshape(B, 1, D)
    return pl.pallas_call(
        _prep_body,
        grid=(B,),
        in_specs=[
            pl.BlockSpec((1, N, D), lambda b: (b, 0, 0)),
            pl.BlockSpec((1, 1, D), lambda b: (b, 0, 0)),
        ],
        out_specs=[
            pl.BlockSpec((1, N, D2), lambda b: (b, 0, 0)),
            pl.BlockSpec((1, N, K), lambda b: (b, 0, 0)),
        ],
        out_shape=[
            jax.ShapeDtypeStruct((B, N, D2), jnp.float32),
            jax.ShapeDtypeStruct((B, N, K), jnp.int32),
        ],
    )(x, f3)


_CHUNK = 128


def _gather_call(table, gidx_flat):
    info = plsc.get_sparse_core_info()
    num_cores = info.num_cores
    nw = num_cores * info.num_subcores
    rpw = _ROWS // nw
    nchunk = rpw // _CHUNK
    mesh = plsc.VectorSubcoreMesh(core_axis_name="c", subcore_axis_name="s")

    @functools.partial(
        pl.kernel,
        mesh=mesh,
        out_type=jax.ShapeDtypeStruct((_ROWS, D2), jnp.float32),
        scratch_types=[
            pltpu.VMEM((_CHUNK,), jnp.int32),
            pltpu.VMEM((_CHUNK, D2), jnp.float32),
            pltpu.SemaphoreType.DMA,
        ],
    )
    def gather_kernel(table_hbm, idx_hbm, out_hbm, idx_v, rows_v, sem):
        wid = lax.axis_index("s") * num_cores + lax.axis_index("c")
        base = wid * rpw

        def body(c, carry):
            off = base + c * _CHUNK
            pltpu.sync_copy(idx_hbm.at[pl.ds(off, _CHUNK)], idx_v)
            pltpu.async_copy(table_hbm.at[idx_v], rows_v, sem).wait()
            pltpu.sync_copy(rows_v, out_hbm.at[pl.ds(off, _CHUNK)])
            return carry

        lax.fori_loop(0, nchunk, body, 0)

    return gather_kernel(table, gidx_flat)


def _ff_body(g_ref, xn_ref, w_ref, b_ref, o_ref):
    wm = w_ref[...]
    ws = wm[:, 0:D2]
    for k in range(1, K):
        ws = ws + wm[:, k * D2:(k + 1) * D2]
    wb = wm.astype(jnp.bfloat16)
    h = lax.dot_general(g_ref[...].astype(jnp.bfloat16), wb,
                        (((1,), (1,)), ((), ())),
                        preferred_element_type=jnp.float32)
    h = h - lax.dot_general(xn_ref[...].astype(jnp.bfloat16),
                            ws.astype(jnp.bfloat16),
                            (((1,), (1,)), ((), ())),
                            preferred_element_type=jnp.float32)
    h = h + b_ref[...]
    a = h[:, :D_MODEL]
    g = h[:, D_MODEL:]
    o_ref[...] = a * jax.nn.sigmoid(g)


def _ff_call(g2, xnf, W, b):
    brow = b.reshape(1, 2 * D_MODEL)
    n_tiles = (B * N) // _FF_TILE
    return pl.pallas_call(
        _ff_body,
        grid=(n_tiles,),
        in_specs=[
            pl.BlockSpec((_FF_TILE, K * D2), lambda i: (i, 0)),
            pl.BlockSpec((_FF_TILE, D2), lambda i: (i, 0)),
            pl.BlockSpec((2 * D_MODEL, K * D2), lambda i: (0, 0)),
            pl.BlockSpec((1, 2 * D_MODEL), lambda i: (0, 0)),
        ],
        out_specs=pl.BlockSpec((_FF_TILE, D_MODEL), lambda i: (i, 0)),
        out_shape=jax.ShapeDtypeStruct((B * N, D_MODEL), jnp.float32),
    )(g2, xnf, W, brow)


def kernel(x, features, W, b):
    xn, gidx = _prep_call(x, features)
    table = xn.reshape(B * N, D2)
    gidx_flat = gidx.reshape(_ROWS)
    g = _gather_call(table, gidx_flat)
    g2 = g.reshape(B * N, K * D2)
    out = _ff_call(g2, table, W, b)
    return out.reshape(B, N, D_MODEL)


# speedup vs baseline: 12.3717x; 1.1088x over previous
import functools

import jax
import jax.numpy as jnp
from jax import lax
from jax.experimental import pallas as pl
from jax.experimental.pallas import tpu as pltpu
from jax.experimental.pallas import tpu_sc as plsc

B, N, D = 16, 1024, 64
K = 8
D_MODEL = 512
D2 = 2 * D

_ROWS = B * N * K
_FF_TILE = 512


def _prep_body(x_ref, f_ref, xn_ref, gidx_ref):
    b = pl.program_id(0)
    xm = x_ref[0]
    fr = f_ref[0]
    mask = fr > 0.1
    x_crd = jnp.where(mask, 0.0, xm)
    x_ftr = jnp.where(mask, xm, 0.0)
    xc = jnp.concatenate([x_crd, x_ftr], axis=1)
    mean = jnp.mean(xc, axis=0, keepdims=True)
    cent = xc - mean
    var = jnp.sum(cent * cent, axis=0, keepdims=True) / (N - 1)
    std = jnp.sqrt(var)
    xn_ref[0] = jnp.clip(cent / (std + 1e-5), -10.0, 10.0)

    sq = jnp.sum(x_crd * x_crd, axis=1, keepdims=True)
    xb = x_crd.astype(jnp.bfloat16)
    dot = lax.dot_general(xb, xb, (((1,), (1,)), ((), ())),
                          preferred_element_type=jnp.float32)
    ones = jnp.ones((1, D), jnp.float32)
    xsq = x_crd * x_crd
    sq_row = lax.dot_general(ones, xsq, (((1,), (1,)), ((), ())),
                             preferred_element_type=jnp.float32,
                             precision=lax.Precision.HIGHEST)
    d2 = sq + sq_row - 2.0 * dot
    dist = jnp.sqrt(jnp.maximum(d2, 1e-12))

    col = lax.broadcasted_iota(jnp.int32, (N, N), 1)
    work = dist
    idxs = []
    for _ in range(K):
        m = jnp.min(work, axis=1, keepdims=True)
        cand = jnp.where(work == m, col, N)
        ik = jnp.min(cand, axis=1, keepdims=True)
        idxs.append(ik)
        work = jnp.where(col == ik, jnp.inf, work)
    gidx_ref[0] = jnp.concatenate(idxs, axis=1) + b * N


def _prep_call(x, features):
    nb = x.shape[0]
    f3 = features.reshape(nb, 1, D)
    return pl.pallas_call(
        _prep_body,
        grid=(nb,),
        in_specs=[
            pl.BlockSpec((1, N, D), lambda b: (b, 0, 0)),
            pl.BlockSpec((1, 1, D), lambda b: (b, 0, 0)),
        ],
        out_specs=[
            pl.BlockSpec((1, N, D2), lambda b: (b, 0, 0)),
            pl.BlockSpec((1, N, K), lambda b: (b, 0, 0)),
        ],
        out_shape=[
            jax.ShapeDtypeStruct((nb, N, D2), jnp.float32),
            jax.ShapeDtypeStruct((nb, N, K), jnp.int32),
        ],
    )(x, f3)


_CHUNK = 128


def _gather_call(table, gidx_flat):
    rows = gidx_flat.shape[0]
    info = plsc.get_sparse_core_info()
    num_cores = info.num_cores
    nw = num_cores * info.num_subcores
    rpw = rows // nw
    nchunk = rpw // _CHUNK
    mesh = plsc.VectorSubcoreMesh(core_axis_name="c", subcore_axis_name="s")

    @functools.partial(
        pl.kernel,
        mesh=mesh,
        out_type=jax.ShapeDtypeStruct((rows, D2), jnp.float32),
        scratch_types=[
            pltpu.VMEM((_CHUNK,), jnp.int32),
            pltpu.VMEM((_CHUNK, D2), jnp.float32),
            pltpu.SemaphoreType.DMA,
        ],
    )
    def gather_kernel(table_hbm, idx_hbm, out_hbm, idx_v, rows_v, sem):
        wid = lax.axis_index("s") * num_cores + lax.axis_index("c")
        base = wid * rpw

        def body(c, carry):
            off = base + c * _CHUNK
            pltpu.sync_copy(idx_hbm.at[pl.ds(off, _CHUNK)], idx_v)
            pltpu.async_copy(table_hbm.at[idx_v], rows_v, sem).wait()
            pltpu.sync_copy(rows_v, out_hbm.at[pl.ds(off, _CHUNK)])
            return carry

        lax.fori_loop(0, nchunk, body, 0)

    return gather_kernel(table, gidx_flat)


def _ff_body(g_ref, xn_ref, w_ref, b_ref, o_ref):
    wm = w_ref[...]
    ws = wm[:, 0:D2]
    for k in range(1, K):
        ws = ws + wm[:, k * D2:(k + 1) * D2]
    wb = wm.astype(jnp.bfloat16)
    h = lax.dot_general(g_ref[...].astype(jnp.bfloat16), wb,
                        (((1,), (1,)), ((), ())),
                        preferred_element_type=jnp.float32)
    h = h - lax.dot_general(xn_ref[...].astype(jnp.bfloat16),
                            ws.astype(jnp.bfloat16),
                            (((1,), (1,)), ((), ())),
                            preferred_element_type=jnp.float32)
    h = h + b_ref[...]
    a = h[:, :D_MODEL]
    g = h[:, D_MODEL:]
    o_ref[...] = a * jax.nn.sigmoid(g)


def _ff_call(g2, xnf, W, b):
    brow = b.reshape(1, 2 * D_MODEL)
    nrows = g2.shape[0]
    n_tiles = nrows // _FF_TILE
    return pl.pallas_call(
        _ff_body,
        grid=(n_tiles,),
        in_specs=[
            pl.BlockSpec((_FF_TILE, K * D2), lambda i: (i, 0)),
            pl.BlockSpec((_FF_TILE, D2), lambda i: (i, 0)),
            pl.BlockSpec((2 * D_MODEL, K * D2), lambda i: (0, 0)),
            pl.BlockSpec((1, 2 * D_MODEL), lambda i: (0, 0)),
        ],
        out_specs=pl.BlockSpec((_FF_TILE, D_MODEL), lambda i: (i, 0)),
        out_shape=jax.ShapeDtypeStruct((nrows, D_MODEL), jnp.float32),
    )(g2, xnf, W, brow)


def kernel(x, features, W, b):
    h = B // 2
    xa, ia = _prep_call(x[:h], features[:h])
    xb, ib = _prep_call(x[h:], features[h:])
    ta = xa.reshape(h * N, D2)
    tb = xb.reshape(h * N, D2)
    ga = _gather_call(ta, ia.reshape(h * N * K))
    gb = _gather_call(tb, ib.reshape(h * N * K))
    oa = _ff_call(ga.reshape(h * N, K * D2), ta, W, b)
    ob = _ff_call(gb.reshape(h * N, K * D2), tb, W, b)
    return jnp.concatenate([oa, ob], axis=0).reshape(B, N, D_MODEL)


# speedup vs baseline: 13.9068x; 1.1241x over previous
import functools

import jax
import jax.numpy as jnp
from jax import lax
from jax.experimental import pallas as pl
from jax.experimental.pallas import tpu as pltpu
from jax.experimental.pallas import tpu_sc as plsc

B, N, D = 16, 1024, 64
K = 8
D_MODEL = 512
D2 = 2 * D

_ROWS = B * N * K
_FF_TILE = 512


def _prep_body(x_ref, f_ref, xn_ref, gidx_ref):
    b = pl.program_id(0)
    xm = x_ref[0]
    fr = f_ref[0]
    mask = fr > 0.1
    x_crd = jnp.where(mask, 0.0, xm)
    x_ftr = jnp.where(mask, xm, 0.0)
    xc = jnp.concatenate([x_crd, x_ftr], axis=1)
    mean = jnp.mean(xc, axis=0, keepdims=True)
    cent = xc - mean
    var = jnp.sum(cent * cent, axis=0, keepdims=True) / (N - 1)
    std = jnp.sqrt(var)
    xn_ref[0] = jnp.clip(cent / (std + 1e-5), -10.0, 10.0)

    sq = jnp.sum(x_crd * x_crd, axis=1, keepdims=True)
    xb = x_crd.astype(jnp.bfloat16)
    dot = lax.dot_general(xb, xb, (((1,), (1,)), ((), ())),
                          preferred_element_type=jnp.float32)
    sq_row = lax.transpose(sq, (1, 0))
    d2 = sq + sq_row - 2.0 * dot
    dist = jnp.sqrt(jnp.maximum(d2, 1e-12))

    colf = lax.broadcasted_iota(jnp.int32, (N, N), 1).astype(jnp.float32)
    work = dist
    idxs = []
    for _ in range(K):
        m = jnp.min(work, axis=1, keepdims=True)
        ikf = jnp.min(jnp.where(work == m, colf, jnp.float32(N)),
                      axis=1, keepdims=True)
        idxs.append(ikf)
        work = jnp.where(colf == ikf, jnp.inf, work)
    gidx = jnp.concatenate(idxs, axis=1).astype(jnp.int32)
    gidx_ref[0] = gidx + b * N


def _prep_call(x, features):
    nb = x.shape[0]
    f3 = features.reshape(nb, 1, D)
    return pl.pallas_call(
        _prep_body,
        grid=(nb,),
        in_specs=[
            pl.BlockSpec((1, N, D), lambda b: (b, 0, 0)),
            pl.BlockSpec((1, 1, D), lambda b: (b, 0, 0)),
        ],
        out_specs=[
            pl.BlockSpec((1, N, D2), lambda b: (b, 0, 0)),
            pl.BlockSpec((1, N, K), lambda b: (b, 0, 0)),
        ],
        out_shape=[
            jax.ShapeDtypeStruct((nb, N, D2), jnp.float32),
            jax.ShapeDtypeStruct((nb, N, K), jnp.int32),
        ],
    )(x, f3)


_CHUNK = 128


def _gather_call(table, gidx_flat):
    rows = gidx_flat.shape[0]
    info = plsc.get_sparse_core_info()
    num_cores = info.num_cores
    nw = num_cores * info.num_subcores
    rpw = rows // nw
    nchunk = rpw // _CHUNK
    mesh = plsc.VectorSubcoreMesh(core_axis_name="c", subcore_axis_name="s")

    @functools.partial(
        pl.kernel,
        mesh=mesh,
        out_type=jax.ShapeDtypeStruct((rows, D2), jnp.float32),
        scratch_types=[
            pltpu.VMEM((_CHUNK,), jnp.int32),
            pltpu.VMEM((_CHUNK, D2), jnp.float32),
            pltpu.SemaphoreType.DMA,
        ],
    )
    def gather_kernel(table_hbm, idx_hbm, out_hbm, idx_v, rows_v, sem):
        wid = lax.axis_index("s") * num_cores + lax.axis_index("c")
        base = wid * rpw

        def body(c, carry):
            off = base + c * _CHUNK
            pltpu.sync_copy(idx_hbm.at[pl.ds(off, _CHUNK)], idx_v)
            pltpu.async_copy(table_hbm.at[idx_v], rows_v, sem).wait()
            pltpu.sync_copy(rows_v, out_hbm.at[pl.ds(off, _CHUNK)])
            return carry

        lax.fori_loop(0, nchunk, body, 0)

    return gather_kernel(table, gidx_flat)


def _ff_body(g_ref, xn_ref, w_ref, b_ref, o_ref):
    wm = w_ref[...]
    ws = wm[:, 0:D2]
    for k in range(1, K):
        ws = ws + wm[:, k * D2:(k + 1) * D2]
    wb = wm.astype(jnp.bfloat16)
    h = lax.dot_general(g_ref[...].astype(jnp.bfloat16), wb,
                        (((1,), (1,)), ((), ())),
                        preferred_element_type=jnp.float32)
    h = h - lax.dot_general(xn_ref[...].astype(jnp.bfloat16),
                            ws.astype(jnp.bfloat16),
                            (((1,), (1,)), ((), ())),
                            preferred_element_type=jnp.float32)
    h = h + b_ref[...]
    a = h[:, :D_MODEL]
    g = h[:, D_MODEL:]
    o_ref[...] = a * jax.nn.sigmoid(g)


def _ff_call(g2, xnf, W, b):
    brow = b.reshape(1, 2 * D_MODEL)
    nrows = g2.shape[0]
    n_tiles = nrows // _FF_TILE
    return pl.pallas_call(
        _ff_body,
        grid=(n_tiles,),
        in_specs=[
            pl.BlockSpec((_FF_TILE, K * D2), lambda i: (i, 0)),
            pl.BlockSpec((_FF_TILE, D2), lambda i: (i, 0)),
            pl.BlockSpec((2 * D_MODEL, K * D2), lambda i: (0, 0)),
            pl.BlockSpec((1, 2 * D_MODEL), lambda i: (0, 0)),
        ],
        out_specs=pl.BlockSpec((_FF_TILE, D_MODEL), lambda i: (i, 0)),
        out_shape=jax.ShapeDtypeStruct((nrows, D_MODEL), jnp.float32),
    )(g2, xnf, W, brow)


def kernel(x, features, W, b):
    h = B // 2
    xa, ia = _prep_call(x[:h], features[:h])
    xb, ib = _prep_call(x[h:], features[h:])
    ta = xa.reshape(h * N, D2)
    tb = xb.reshape(h * N, D2)
    ga = _gather_call(ta, ia.reshape(h * N * K))
    gb = _gather_call(tb, ib.reshape(h * N * K))
    oa = _ff_call(ga.reshape(h * N, K * D2), ta, W, b)
    ob = _ff_call(gb.reshape(h * N, K * D2), tb, W, b)
    return jnp.concatenate([oa, ob], axis=0).reshape(B, N, D_MODEL)
